# baseline (speedup 1.0000x reference)
import functools

import jax
import jax.numpy as jnp
from jax.experimental import pallas as pl
from jax.experimental.pallas import tpu as pltpu
from jax.experimental.pallas import tpu_sc as plsc

B = 8
L = 2048
D = 128
WINDOW = 9
K_SYM = 5
PATCH_HIGH = 3
PATCH_LOW = 12
ENT_THR_ROBUST = 1.51
NCAND = (L + PATCH_HIGH - 1) // PATCH_HIGH
KP = 704
CSLEN = 2080


def _ent_body(xp_ref, ent_ref, cs_ref):
    xp = xp_ref[...]
    counts = []
    for s in range(K_SYM):
        ind = (xp == s).astype(jnp.float32)
        c = ind[:, 4:4 + L]
        for w in range(WINDOW):
            if w != 4:
                c = c + ind[:, w:w + L]
        counts.append(c)
    total = counts[0] + counts[1] + counts[2] + counts[3] + counts[4]
    total = jnp.maximum(total, 1e-12)
    ent = jnp.zeros((B, L), jnp.float32)
    for s in range(K_SYM):
        p = counts[s] / total
        ent = ent - p * jnp.log2(p + 1e-12)
    ent_ref[...] = ent

    xf = xp[:, 4:4 + L].astype(jnp.float32)
    inc = xf
    sh = 1
    while sh < L:
        z = jnp.zeros((B, sh), jnp.float32)
        inc = inc + jnp.concatenate([z, inc[:, :L - sh]], axis=1)
        sh *= 2
    zcol = jnp.zeros((B, 1), jnp.float32)
    ztail = jnp.zeros((B, CSLEN - L - 1), jnp.float32)
    cs_ref[...] = jnp.concatenate([zcol, inc, ztail], axis=1)


def _entropy_cs(xp):
    return pl.pallas_call(
        _ent_body,
        out_shape=[
            jax.ShapeDtypeStruct((B, L), jnp.float32),
            jax.ShapeDtypeStruct((B, CSLEN), jnp.float32),
        ],
    )(xp)


def _walk_patches(entropy, cs):
    mesh = plsc.VectorSubcoreMesh(core_axis_name="c", subcore_axis_name="s")
    nworkers = 32

    @functools.partial(
        pl.kernel,
        out_type=[
            jax.ShapeDtypeStruct((nworkers, KP), jnp.float32),
            jax.ShapeDtypeStruct((nworkers, 16), jnp.float32),
        ],
        mesh=mesh,
        scratch_types=[
            pltpu.VMEM((L + 32,), jnp.float32),
            pltpu.VMEM((CSLEN,), jnp.float32),
            pltpu.VMEM((KP,), jnp.float32),
            pltpu.VMEM((16,), jnp.float32),
        ],
    )
    def walk(ent_hbm, cs_hbm, pm_hbm, cnt_hbm, ent_v, cs_v, pm_v, cnt_v):
        wid = jax.lax.axis_index("s") * 2 + jax.lax.axis_index("c")
        row = jax.lax.rem(wid, B)
        pltpu.sync_copy(ent_hbm.at[row], ent_v.at[pl.ds(0, L)])
        pltpu.sync_copy(cs_hbm.at[row], cs_v)
        lane_i = jax.lax.iota(jnp.int32, 16)
        fL = float(L)

        def body(_, carry):
            i, t, cs_i, pmvec = carry
            active = i < L
            act_f = jnp.where(active, 1.0, 0.0)
            hi = ent_v[pl.ds(i, 16)][0] > ENT_THR_ROBUST
            psize = jnp.where(hi, PATCH_HIGH, PATCH_LOW)
            j = jnp.minimum(i + psize, L)
            cs_j = cs_v[pl.ds(j, 16)][0]
            den = jnp.maximum(j - i, 1)
            rden = jnp.where(
                den == 12, 1.0 / 12.0,
                jnp.where(den == 11, 1.0 / 11.0,
                          jnp.where(den == 8, 0.125,
                                    jnp.where(den == 5, 0.2,
                                              jnp.where(den == 3, 1.0 / 3.0,
                                                        jnp.where(den == 2,
                                                                  0.5,
                                                                  1.0))))))
            pm = (cs_j - cs_i) * rden
            tm = jnp.bitwise_and(t, 15)
            keep = jnp.minimum(tm, 1).astype(jnp.float32)
            d = lane_i - tm
            eq_f = (1 - jnp.minimum(d * d, 1)).astype(jnp.float32)
            pmvec2 = pmvec * keep + eq_f * (pm * act_f)
            base = jnp.bitwise_and(t, -16)
            pm_v[pl.ds(base, 16)] = pmvec2
            i2 = jnp.where(active, j, i)
            cs_i2 = jnp.where(active, cs_j, cs_i)
            t2 = t + jnp.where(active, 1, 0)
            return (i2, t2, cs_i2, pmvec2)

        init = (jnp.int32(0), jnp.int32(0), jnp.float32(0.0),
                jnp.zeros((16,), jnp.float32))
        final = jax.lax.fori_loop(0, NCAND, body, init)
        cnt_v[...] = jnp.full((16,), 1.0, jnp.float32) * final[1].astype(
            jnp.float32)
        pltpu.sync_copy(pm_v, pm_hbm.at[wid])
        pltpu.sync_copy(cnt_v, cnt_hbm.at[wid])

    pm, cnt = walk(entropy, cs)
    return pm[:B], cnt[:B, 0:1]


def _feat_body(pm_ref, cnt_ref, w1_ref, b1_ref, w2_ref, b2_ref, out_ref):
    pm = pm_ref[...]
    count = cnt_ref[...]
    tt = jax.lax.broadcasted_iota(jnp.int32, (B, KP), 1).astype(jnp.float32)
    msk = (tt < count).astype(jnp.float32)
    w1 = w1_ref[...]
    b1 = b1_ref[...]
    h = jnp.maximum(pm[:, :, None] * w1 + b1, 0.0) * msk[:, :, None]
    s_h = jnp.sum(h, axis=1)
    out = jax.lax.dot_general(
        s_h, w2_ref[...], (((1,), (0,)), ((), ())),
        preferred_element_type=jnp.float32,
    )
    out_ref[...] = out / count + b2_ref[...]


def _features(pm, cnt, W1, b1, W2, b2):
    return pl.pallas_call(
        _feat_body,
        out_shape=jax.ShapeDtypeStruct((B, D), jnp.float32),
    )(pm, cnt, W1, b1.reshape(1, D), W2, b2.reshape(1, D))


def kernel(x, W1, b1, W2, b2):
    xp = jnp.pad(x, ((0, 0), (4, 4)), constant_values=-1)
    entropy, cs = _entropy_cs(xp)
    pm, cnt = _walk_patches(entropy, cs)
    blt = _features(pm, cnt, W1, b1, W2, b2)
    return (blt, entropy)


# speedup vs baseline: 1.2135x; 1.2135x over previous
import functools

import jax
import jax.numpy as jnp
from jax.experimental import pallas as pl
from jax.experimental.pallas import tpu as pltpu
from jax.experimental.pallas import tpu_sc as plsc

B = 8
L = 2048
D = 128
WINDOW = 9
K_SYM = 5
PATCH_HIGH = 3
PATCH_LOW = 12
ENT_THR_ROBUST = 1.51
NCAND = (L + PATCH_HIGH - 1) // PATCH_HIGH
KP = 704
CSLEN = 2080


def _ent_body(xp_ref, ent_ref, cs_ref):
    xp = xp_ref[...]
    counts = []
    for s in range(K_SYM):
        ind = (xp == s).astype(jnp.float32)
        c = ind[:, 4:4 + L]
        for w in range(WINDOW):
            if w != 4:
                c = c + ind[:, w:w + L]
        counts.append(c)
    total = counts[0] + counts[1] + counts[2] + counts[3] + counts[4]
    total = jnp.maximum(total, 1e-12)
    ent = jnp.zeros((B, L), jnp.float32)
    for s in range(K_SYM):
        p = counts[s] / total
        ent = ent - p * jnp.log2(p + 1e-12)
    ent_ref[...] = ent

    xf = xp[:, 4:4 + L].astype(jnp.float32)
    inc = xf
    sh = 1
    while sh < L:
        z = jnp.zeros((B, sh), jnp.float32)
        inc = inc + jnp.concatenate([z, inc[:, :L - sh]], axis=1)
        sh *= 2
    zcol = jnp.zeros((B, 1), jnp.float32)
    ztail = jnp.zeros((B, CSLEN - L - 1), jnp.float32)
    cs_ref[...] = jnp.concatenate([zcol, inc, ztail], axis=1)


def _entropy_cs(xp):
    return pl.pallas_call(
        _ent_body,
        out_shape=[
            jax.ShapeDtypeStruct((B, L), jnp.float32),
            jax.ShapeDtypeStruct((B, CSLEN), jnp.float32),
        ],
    )(xp)


def _walk_patches(entropy, cs):
    mesh = plsc.VectorSubcoreMesh(core_axis_name="c", subcore_axis_name="s")
    nworkers = 32

    @functools.partial(
        pl.kernel,
        out_type=[
            jax.ShapeDtypeStruct((nworkers, KP), jnp.float32),
            jax.ShapeDtypeStruct((nworkers, 16), jnp.float32),
        ],
        mesh=mesh,
        scratch_types=[
            pltpu.VMEM((L + 48,), jnp.float32),
            pltpu.VMEM((CSLEN,), jnp.float32),
            pltpu.VMEM((KP,), jnp.float32),
            pltpu.VMEM((16,), jnp.float32),
        ],
    )
    def walk(ent_hbm, cs_hbm, pm_hbm, cnt_hbm, ent_v, cs_v, pm_v, cnt_v):
        wid = jax.lax.axis_index("s") * 2 + jax.lax.axis_index("c")
        row = jax.lax.rem(wid, B)
        pltpu.sync_copy(ent_hbm.at[row], ent_v.at[pl.ds(0, L)])
        pltpu.sync_copy(cs_hbm.at[row], cs_v)
        lane_i = jax.lax.iota(jnp.int32, 16)

        def body(_, carry):
            i, t, last, e_cur, cs_cur, pmvec = carry
            active = i < L
            act_f = jnp.where(active, 1.0, 0.0)
            hi = e_cur > ENT_THR_ROBUST
            i3 = i + PATCH_HIGH
            i12 = i + PATCH_LOW
            j3 = jnp.minimum(i3, L)
            j12 = jnp.minimum(i12, L)
            e3 = ent_v[pl.ds(i3, 16)][0]
            e12 = ent_v[pl.ds(i12, 16)][0]
            c3 = cs_v[pl.ds(j3, 16)][0]
            c12 = cs_v[pl.ds(j12, 16)][0]
            j = jnp.where(hi, j3, j12)
            cs_j = jnp.where(hi, c3, c12)
            rden = jnp.where(hi, 1.0 / PATCH_HIGH, 1.0 / PATCH_LOW)
            pm = (cs_j - cs_cur) * rden
            tm = jnp.bitwise_and(t, 15)
            keep = jnp.minimum(tm, 1).astype(jnp.float32)
            d = lane_i - tm
            eq_f = (1 - jnp.minimum(d * d, 1)).astype(jnp.float32)
            pmvec2 = pmvec * keep + eq_f * (pm * act_f)
            base = jnp.bitwise_and(t, -16)
            pm_v[pl.ds(base, 16)] = pmvec2
            i2 = jnp.where(active, j, i)
            last2 = jnp.where(active, i, last)
            e2 = jnp.where(active, jnp.where(hi, e3, e12), e_cur)
            cs2 = jnp.where(active, cs_j, cs_cur)
            t2 = t + jnp.where(active, 1, 0)
            return (i2, t2, last2, e2, cs2, pmvec2)

        e0 = ent_v[pl.ds(0, 16)][0]
        init = (jnp.int32(0), jnp.int32(0), jnp.int32(0), e0,
                jnp.float32(0.0), jnp.zeros((16,), jnp.float32))
        final = jax.lax.fori_loop(0, NCAND, body, init)
        count = final[1]
        i_last = final[2]
        cs_total = cs_v[pl.ds(L, 16)][0]
        cs_il = cs_v[pl.ds(i_last, 16)][0]
        def lane(k):
            dk = lane_i - k
            return (1 - jnp.minimum(dk * dk, 1)).astype(jnp.float32)
        cnt_v[...] = (lane(0) * count.astype(jnp.float32)
                      + lane(1) * i_last.astype(jnp.float32)
                      + lane(2) * cs_total + lane(3) * cs_il)
        pltpu.sync_copy(pm_v, pm_hbm.at[wid])
        pltpu.sync_copy(cnt_v, cnt_hbm.at[wid])

    pm, cnt = walk(entropy, cs)
    return pm[:B], cnt[:B, 0:8]


def _feat_body(pm_ref, cnt_ref, w1_ref, b1_ref, w2_ref, b2_ref, out_ref):
    pm = pm_ref[...]
    count = cnt_ref[:, 0:1]
    i_last = cnt_ref[:, 1:2]
    cs_total = cnt_ref[:, 2:3]
    cs_il = cnt_ref[:, 3:4]
    den_last = jnp.maximum(float(L) - i_last, 1.0)
    pm_last = (cs_total - cs_il) / den_last
    tt = jax.lax.broadcasted_iota(jnp.int32, (B, KP), 1).astype(jnp.float32)
    pm = jnp.where(tt == count - 1.0, pm_last, pm)
    msk = (tt < count).astype(jnp.float32)
    w1 = w1_ref[...]
    b1 = b1_ref[...]
    h = jnp.maximum(pm[:, :, None] * w1 + b1, 0.0) * msk[:, :, None]
    s_h = jnp.sum(h, axis=1)
    out = jax.lax.dot_general(
        s_h, w2_ref[...], (((1,), (0,)), ((), ())),
        preferred_element_type=jnp.float32,
    )
    out_ref[...] = out / count + b2_ref[...]


def _features(pm, cnt, W1, b1, W2, b2):
    return pl.pallas_call(
        _feat_body,
        out_shape=jax.ShapeDtypeStruct((B, D), jnp.float32),
    )(pm, cnt, W1, b1.reshape(1, D), W2, b2.reshape(1, D))


def kernel(x, W1, b1, W2, b2):
    xp = jnp.pad(x, ((0, 0), (4, 4)), constant_values=-1)
    entropy, cs = _entropy_cs(xp)
    pm, cnt = _walk_patches(entropy, cs)
    blt = _features(pm, cnt, W1, b1, W2, b2)
    return (blt, entropy)


# speedup vs baseline: 1.2690x; 1.0457x over previous
import functools

import jax
import jax.numpy as jnp
from jax.experimental import pallas as pl
from jax.experimental.pallas import tpu as pltpu
from jax.experimental.pallas import tpu_sc as plsc

B = 8
L = 2048
D = 128
WINDOW = 9
K_SYM = 5
PATCH_HIGH = 3
PATCH_LOW = 12
ENT_THR_ROBUST = 1.51
NCAND = (L + PATCH_HIGH - 1) // PATCH_HIGH
KP = 704
CSLEN = 2080


def _ent_body(xp_ref, ent_ref, cs_ref):
    xp = xp_ref[...]
    counts = []
    for s in range(K_SYM):
        ind = (xp == s).astype(jnp.float32)
        c = ind[:, 4:4 + L]
        for w in range(WINDOW):
            if w != 4:
                c = c + ind[:, w:w + L]
        counts.append(c)
    total = counts[0] + counts[1] + counts[2] + counts[3] + counts[4]
    total = jnp.maximum(total, 1e-12)
    ent = jnp.zeros((B, L), jnp.float32)
    for s in range(K_SYM):
        p = counts[s] / total
        ent = ent - p * jnp.log2(p + 1e-12)
    ent_ref[...] = ent

    xf = xp[:, 4:4 + L].astype(jnp.float32)
    inc = xf
    sh = 1
    while sh < L:
        z = jnp.zeros((B, sh), jnp.float32)
        inc = inc + jnp.concatenate([z, inc[:, :L - sh]], axis=1)
        sh *= 2
    zcol = jnp.zeros((B, 1), jnp.float32)
    ztail = jnp.zeros((B, CSLEN - L - 1), jnp.float32)
    cs_ref[...] = jnp.concatenate([zcol, inc, ztail], axis=1)


def _entropy_cs(xp):
    return pl.pallas_call(
        _ent_body,
        out_shape=[
            jax.ShapeDtypeStruct((B, L), jnp.float32),
            jax.ShapeDtypeStruct((B, CSLEN), jnp.float32),
        ],
    )(xp)


def _walk_patches(entropy, cs):
    mesh = plsc.VectorSubcoreMesh(core_axis_name="c", subcore_axis_name="s")
    nworkers = 32

    @functools.partial(
        pl.kernel,
        out_type=[
            jax.ShapeDtypeStruct((nworkers, KP), jnp.float32),
            jax.ShapeDtypeStruct((nworkers, 16), jnp.float32),
        ],
        mesh=mesh,
        scratch_types=[
            pltpu.VMEM((L + 48,), jnp.float32),
            pltpu.VMEM((CSLEN,), jnp.float32),
            pltpu.VMEM((KP,), jnp.float32),
            pltpu.VMEM((16,), jnp.float32),
        ],
    )
    def walk(ent_hbm, cs_hbm, pm_hbm, cnt_hbm, ent_v, cs_v, pm_v, cnt_v):
        wid = jax.lax.axis_index("s") * 2 + jax.lax.axis_index("c")
        row = jax.lax.rem(wid, B)
        pltpu.sync_copy(ent_hbm.at[row], ent_v.at[pl.ds(0, L)])
        pltpu.sync_copy(cs_hbm.at[row], cs_v)
        zero16 = jnp.zeros((16,), jnp.float32)
        ent_v[pl.ds(L, 16)] = zero16
        ent_v[pl.ds(L + 16, 16)] = zero16
        ent_v[pl.ds(L + 32, 16)] = zero16
        lane_i = jax.lax.iota(jnp.int32, 16)

        def body(_, carry):
            i, t, last, e_cur, cs_cur = carry
            active = i < L
            hi = e_cur > ENT_THR_ROBUST
            i3 = i + PATCH_HIGH
            i12 = i + PATCH_LOW
            j3 = jnp.minimum(i3, L)
            j12 = jnp.minimum(i12, L)
            e3 = ent_v[pl.ds(i3, 16)][0]
            e12 = ent_v[pl.ds(i12, 16)][0]
            c3 = cs_v[pl.ds(j3, 16)][0]
            c12 = cs_v[pl.ds(j12, 16)][0]
            j = jnp.where(hi, j3, j12)
            cs_j = jnp.where(hi, c3, c12)
            rden = jnp.where(hi, 1.0 / PATCH_HIGH, 1.0 / PATCH_LOW)
            pm = (cs_j - cs_cur) * rden
            pm_v[pl.ds(t, 16)] = jnp.broadcast_to(pm, (16,))
            i2 = jnp.where(active, j, i)
            last2 = jnp.where(active, i, last)
            e2 = jnp.where(active, jnp.where(hi, e3, e12), e_cur)
            cs2 = jnp.where(active, cs_j, cs_cur)
            t2 = t + jnp.where(active, 1, 0)
            return (i2, t2, last2, e2, cs2)

        e0 = ent_v[pl.ds(0, 16)][0]
        init = (jnp.int32(0), jnp.int32(0), jnp.int32(0), e0,
                jnp.float32(0.0))
        final = jax.lax.fori_loop(0, NCAND, body, init, unroll=4)
        count = final[1]
        i_last = final[2]
        cs_total = cs_v[pl.ds(L, 16)][0]
        cs_il = cs_v[pl.ds(i_last, 16)][0]
        def lane(k):
            dk = lane_i - k
            return (1 - jnp.minimum(dk * dk, 1)).astype(jnp.float32)
        cnt_v[...] = (lane(0) * count.astype(jnp.float32)
                      + lane(1) * i_last.astype(jnp.float32)
                      + lane(2) * cs_total + lane(3) * cs_il)
        pltpu.sync_copy(pm_v, pm_hbm.at[wid])
        pltpu.sync_copy(cnt_v, cnt_hbm.at[wid])

    pm, cnt = walk(entropy, cs)
    return pm[:B], cnt[:B, 0:8]


def _feat_body(pm_ref, cnt_ref, w1_ref, b1_ref, w2_ref, b2_ref, out_ref):
    pm = pm_ref[...]
    count = cnt_ref[:, 0:1]
    i_last = cnt_ref[:, 1:2]
    cs_total = cnt_ref[:, 2:3]
    cs_il = cnt_ref[:, 3:4]
    den_last = jnp.maximum(float(L) - i_last, 1.0)
    pm_last = (cs_total - cs_il) / den_last
    tt = jax.lax.broadcasted_iota(jnp.int32, (B, KP), 1).astype(jnp.float32)
    pm = jnp.where(tt == count - 1.0, pm_last, pm)
    msk = (tt < count).astype(jnp.float32)
    w1 = w1_ref[...]
    b1 = b1_ref[...]
    h = jnp.maximum(pm[:, :, None] * w1 + b1, 0.0) * msk[:, :, None]
    s_h = jnp.sum(h, axis=1)
    out = jax.lax.dot_general(
        s_h, w2_ref[...], (((1,), (0,)), ((), ())),
        preferred_element_type=jnp.float32,
    )
    out_ref[...] = out / count + b2_ref[...]


def _features(pm, cnt, W1, b1, W2, b2):
    return pl.pallas_call(
        _feat_body,
        out_shape=jax.ShapeDtypeStruct((B, D), jnp.float32),
    )(pm, cnt, W1, b1.reshape(1, D), W2, b2.reshape(1, D))


def kernel(x, W1, b1, W2, b2):
    xp = jnp.pad(x, ((0, 0), (4, 4)), constant_values=-1)
    entropy, cs = _entropy_cs(xp)
    pm, cnt = _walk_patches(entropy, cs)
    blt = _features(pm, cnt, W1, b1, W2, b2)
    return (blt, entropy)


# speedup vs baseline: 1.3413x; 1.0570x over previous
import functools

import jax
import jax.numpy as jnp
from jax.experimental import pallas as pl
from jax.experimental.pallas import tpu as pltpu
from jax.experimental.pallas import tpu_sc as plsc

B = 8
L = 2048
D = 128
WINDOW = 9
K_SYM = 5
PATCH_HIGH = 3
PATCH_LOW = 12
ENT_THR_ROBUST = 1.51
NCAND = (L + PATCH_HIGH - 1) // PATCH_HIGH
KP = 704
CSLEN = 2080


def _ent_body(xp_ref, ent_ref, cs_ref):
    xp = xp_ref[...]
    counts = []
    for s in range(K_SYM):
        ind = (xp == s).astype(jnp.float32)
        c = ind[:, 4:4 + L]
        for w in range(WINDOW):
            if w != 4:
                c = c + ind[:, w:w + L]
        counts.append(c)
    total = counts[0] + counts[1] + counts[2] + counts[3] + counts[4]
    total = jnp.maximum(total, 1e-12)
    ent = jnp.zeros((B, L), jnp.float32)
    for s in range(K_SYM):
        p = counts[s] / total
        ent = ent - p * jnp.log2(p + 1e-12)
    ent_ref[...] = ent

    xf = xp[:, 4:4 + L].astype(jnp.float32)
    inc = xf
    sh = 1
    while sh < L:
        z = jnp.zeros((B, sh), jnp.float32)
        inc = inc + jnp.concatenate([z, inc[:, :L - sh]], axis=1)
        sh *= 2
    zcol = jnp.zeros((B, 1), jnp.float32)
    ztail = jnp.zeros((B, CSLEN - L - 1), jnp.float32)
    cs_ref[...] = jnp.concatenate([zcol, inc, ztail], axis=1)


def _entropy_cs(xp):
    return pl.pallas_call(
        _ent_body,
        out_shape=[
            jax.ShapeDtypeStruct((B, L), jnp.float32),
            jax.ShapeDtypeStruct((B, CSLEN), jnp.float32),
        ],
    )(xp)


def _walk_patches(entropy, cs):
    mesh = plsc.VectorSubcoreMesh(core_axis_name="c", subcore_axis_name="s",
                                  num_cores=1)
    nworkers = 16

    @functools.partial(
        pl.kernel,
        out_type=[
            jax.ShapeDtypeStruct((nworkers, KP), jnp.float32),
            jax.ShapeDtypeStruct((nworkers, 16), jnp.float32),
        ],
        mesh=mesh,
        scratch_types=[
            pltpu.VMEM((L + 48,), jnp.float32),
            pltpu.VMEM((CSLEN,), jnp.float32),
            pltpu.VMEM((KP,), jnp.float32),
            pltpu.VMEM((16,), jnp.float32),
        ],
    )
    def walk(ent_hbm, cs_hbm, pm_hbm, cnt_hbm, ent_v, cs_v, pm_v, cnt_v):
        wid = jax.lax.axis_index("s") + jax.lax.axis_index("c")
        row = jax.lax.rem(wid, B)
        pltpu.sync_copy(ent_hbm.at[row], ent_v.at[pl.ds(0, L)])
        pltpu.sync_copy(cs_hbm.at[row], cs_v)
        zero16 = jnp.zeros((16,), jnp.float32)
        ent_v[pl.ds(L, 16)] = zero16
        ent_v[pl.ds(L + 16, 16)] = zero16
        ent_v[pl.ds(L + 32, 16)] = zero16
        lane_i = jax.lax.iota(jnp.int32, 16)

        def body(_, carry):
            i, t, last, e_cur, cs_cur = carry
            active = i < L
            hi = e_cur > ENT_THR_ROBUST
            i3 = i + PATCH_HIGH
            i12 = i + PATCH_LOW
            j3 = jnp.minimum(i3, L)
            j12 = jnp.minimum(i12, L)
            e3 = ent_v[pl.ds(i3, 16)][0]
            e12 = ent_v[pl.ds(i12, 16)][0]
            c3 = cs_v[pl.ds(j3, 16)][0]
            c12 = cs_v[pl.ds(j12, 16)][0]
            j = jnp.where(hi, j3, j12)
            cs_j = jnp.where(hi, c3, c12)
            rden = jnp.where(hi, 1.0 / PATCH_HIGH, 1.0 / PATCH_LOW)
            pm = (cs_j - cs_cur) * rden
            pm_v[pl.ds(t, 16)] = jnp.broadcast_to(pm, (16,))
            i2 = jnp.where(active, j, i)
            last2 = jnp.where(active, i, last)
            e2 = jnp.where(active, jnp.where(hi, e3, e12), e_cur)
            cs2 = jnp.where(active, cs_j, cs_cur)
            t2 = t + jnp.where(active, 1, 0)
            return (i2, t2, last2, e2, cs2)

        e0 = ent_v[pl.ds(0, 16)][0]
        init = (jnp.int32(0), jnp.int32(0), jnp.int32(0), e0,
                jnp.float32(0.0))
        final = jax.lax.fori_loop(0, NCAND, body, init, unroll=8)
        count = final[1]
        i_last = final[2]
        cs_total = cs_v[pl.ds(L, 16)][0]
        cs_il = cs_v[pl.ds(i_last, 16)][0]
        def lane(k):
            dk = lane_i - k
            return (1 - jnp.minimum(dk * dk, 1)).astype(jnp.float32)
        cnt_v[...] = (lane(0) * count.astype(jnp.float32)
                      + lane(1) * i_last.astype(jnp.float32)
                      + lane(2) * cs_total + lane(3) * cs_il)
        pltpu.sync_copy(pm_v, pm_hbm.at[wid])
        pltpu.sync_copy(cnt_v, cnt_hbm.at[wid])

    pm, cnt = walk(entropy, cs)
    return pm[:B], cnt[:B, 0:8]


def _feat_body(pm_ref, cnt_ref, w1_ref, b1_ref, w2_ref, b2_ref, out_ref):
    pm = pm_ref[...]
    count = cnt_ref[:, 0:1]
    i_last = cnt_ref[:, 1:2]
    cs_total = cnt_ref[:, 2:3]
    cs_il = cnt_ref[:, 3:4]
    den_last = jnp.maximum(float(L) - i_last, 1.0)
    pm_last = (cs_total - cs_il) / den_last
    tt = jax.lax.broadcasted_iota(jnp.int32, (B, KP), 1).astype(jnp.float32)
    pm = jnp.where(tt == count - 1.0, pm_last, pm)
    msk = (tt < count).astype(jnp.float32)
    w1 = w1_ref[...]
    b1 = b1_ref[...]
    h = jnp.maximum(pm[:, :, None] * w1 + b1, 0.0) * msk[:, :, None]
    s_h = jnp.sum(h, axis=1)
    out = jax.lax.dot_general(
        s_h, w2_ref[...], (((1,), (0,)), ((), ())),
        preferred_element_type=jnp.float32,
    )
    out_ref[...] = out / count + b2_ref[...]


def _features(pm, cnt, W1, b1, W2, b2):
    return pl.pallas_call(
        _feat_body,
        out_shape=jax.ShapeDtypeStruct((B, D), jnp.float32),
    )(pm, cnt, W1, b1.reshape(1, D), W2, b2.reshape(1, D))


def kernel(x, W1, b1, W2, b2):
    xp = jnp.pad(x, ((0, 0), (4, 4)), constant_values=-1)
    entropy, cs = _entropy_cs(xp)
    pm, cnt = _walk_patches(entropy, cs)
    blt = _features(pm, cnt, W1, b1, W2, b2)
    return (blt, entropy)


# speedup vs baseline: 1.4628x; 1.0906x over previous
import functools

import jax
import jax.numpy as jnp
from jax.experimental import pallas as pl
from jax.experimental.pallas import tpu as pltpu
from jax.experimental.pallas import tpu_sc as plsc

B = 8
L = 2048
D = 128
WINDOW = 9
K_SYM = 5
PATCH_HIGH = 3
PATCH_LOW = 12
ENT_THR_ROBUST = 1.51
NCAND = (L + PATCH_HIGH - 1) // PATCH_HIGH
KP = 704
CSLEN = 2080


def _ent_body(x_ref, ent_ref, cs_ref):
    x = x_ref[...]
    z4 = jnp.zeros((B, WINDOW // 2), jnp.float32)
    counts = []
    for s in range(K_SYM):
        ind = jnp.concatenate(
            [z4, (x == s).astype(jnp.float32), z4], axis=1)
        c = ind[:, 4:4 + L]
        for w in range(WINDOW):
            if w != 4:
                c = c + ind[:, w:w + L]
        counts.append(c)
    total = counts[0] + counts[1] + counts[2] + counts[3] + counts[4]
    total = jnp.maximum(total, 1e-12)
    ent = jnp.zeros((B, L), jnp.float32)
    for s in range(K_SYM):
        p = counts[s] / total
        ent = ent - p * jnp.log2(p + 1e-12)
    ent_ref[...] = ent

    xf = x.astype(jnp.float32)
    inc = xf
    sh = 1
    while sh < L:
        z = jnp.zeros((B, sh), jnp.float32)
        inc = inc + jnp.concatenate([z, inc[:, :L - sh]], axis=1)
        sh *= 2
    zcol = jnp.zeros((B, 1), jnp.float32)
    ztail = jnp.zeros((B, CSLEN - L - 1), jnp.float32)
    cs_ref[...] = jnp.concatenate([zcol, inc, ztail], axis=1)


def _entropy_cs(x):
    return pl.pallas_call(
        _ent_body,
        out_shape=[
            jax.ShapeDtypeStruct((B, L), jnp.float32),
            jax.ShapeDtypeStruct((B, CSLEN), jnp.float32),
        ],
    )(x)


def _walk_patches(entropy, cs):
    mesh = plsc.VectorSubcoreMesh(core_axis_name="c", subcore_axis_name="s",
                                  num_cores=1)
    nworkers = 16

    @functools.partial(
        pl.kernel,
        out_type=[
            jax.ShapeDtypeStruct((B, KP), jnp.float32),
            jax.ShapeDtypeStruct((B, 16), jnp.float32),
        ],
        mesh=mesh,
        scratch_types=[
            pltpu.VMEM((L + 48,), jnp.float32),
            pltpu.VMEM((CSLEN,), jnp.float32),
            pltpu.VMEM((KP,), jnp.float32),
            pltpu.VMEM((16,), jnp.float32),
        ],
    )
    def walk(ent_hbm, cs_hbm, pm_hbm, cnt_hbm, ent_v, cs_v, pm_v, cnt_v):
        wid = jax.lax.axis_index("s") + jax.lax.axis_index("c")
        row = jax.lax.rem(wid, B)
        pltpu.sync_copy(ent_hbm.at[row], ent_v.at[pl.ds(0, L)])
        pltpu.sync_copy(cs_hbm.at[row], cs_v)
        zero16 = jnp.zeros((16,), jnp.float32)
        ent_v[pl.ds(L, 16)] = zero16
        ent_v[pl.ds(L + 16, 16)] = zero16
        ent_v[pl.ds(L + 32, 16)] = zero16
        lane_i = jax.lax.iota(jnp.int32, 16)

        def body(_, carry):
            i, t, last, e_cur, cs_cur = carry
            active = i < L
            hi = e_cur > ENT_THR_ROBUST
            i3 = i + PATCH_HIGH
            i12 = i + PATCH_LOW
            j3 = jnp.minimum(i3, L)
            j12 = jnp.minimum(i12, L)
            e3 = ent_v[pl.ds(i3, 16)][0]
            e12 = ent_v[pl.ds(i12, 16)][0]
            c3 = cs_v[pl.ds(j3, 16)][0]
            c12 = cs_v[pl.ds(j12, 16)][0]
            j = jnp.where(hi, j3, j12)
            cs_j = jnp.where(hi, c3, c12)
            rden = jnp.where(hi, 1.0 / PATCH_HIGH, 1.0 / PATCH_LOW)
            pm = (cs_j - cs_cur) * rden
            pm_v[pl.ds(t, 16)] = jnp.broadcast_to(pm, (16,))
            i2 = jnp.where(active, j, i)
            last2 = jnp.where(active, i, last)
            e2 = jnp.where(active, jnp.where(hi, e3, e12), e_cur)
            cs2 = jnp.where(active, cs_j, cs_cur)
            t2 = t + jnp.where(active, 1, 0)
            return (i2, t2, last2, e2, cs2)

        e0 = ent_v[pl.ds(0, 16)][0]
        init = (jnp.int32(0), jnp.int32(0), jnp.int32(0), e0,
                jnp.float32(0.0))
        final = jax.lax.fori_loop(0, NCAND, body, init, unroll=8)
        count = final[1]
        i_last = final[2]
        cs_total = cs_v[pl.ds(L, 16)][0]
        cs_il = cs_v[pl.ds(i_last, 16)][0]
        def lane(k):
            dk = lane_i - k
            return (1 - jnp.minimum(dk * dk, 1)).astype(jnp.float32)
        cnt_v[...] = (lane(0) * count.astype(jnp.float32)
                      + lane(1) * i_last.astype(jnp.float32)
                      + lane(2) * cs_total + lane(3) * cs_il)

        @pl.when(wid < B)
        def _():
            pltpu.sync_copy(pm_v, pm_hbm.at[wid])
            pltpu.sync_copy(cnt_v, cnt_hbm.at[wid])

    return walk(entropy, cs)


def _feat_body(pm_ref, cnt_ref, w1_ref, b1_ref, w2_ref, b2_ref, out_ref):
    pm = pm_ref[...]
    count = cnt_ref[:, 0:1]
    i_last = cnt_ref[:, 1:2]
    cs_total = cnt_ref[:, 2:3]
    cs_il = cnt_ref[:, 3:4]
    den_last = jnp.maximum(float(L) - i_last, 1.0)
    pm_last = (cs_total - cs_il) / den_last
    tt = jax.lax.broadcasted_iota(jnp.int32, (B, KP), 1).astype(jnp.float32)
    pm = jnp.where(tt == count - 1.0, pm_last, pm)
    msk = (tt < count).astype(jnp.float32)
    w1 = w1_ref[...]
    b1 = b1_ref[...]
    h = jnp.maximum(pm[:, :, None] * w1 + b1, 0.0) * msk[:, :, None]
    s_h = jnp.sum(h, axis=1)
    out = jax.lax.dot_general(
        s_h, w2_ref[...], (((1,), (0,)), ((), ())),
        preferred_element_type=jnp.float32,
    )
    out_ref[...] = out / count + b2_ref[...]


def _features(pm, cnt, W1, b1, W2, b2):
    return pl.pallas_call(
        _feat_body,
        out_shape=jax.ShapeDtypeStruct((B, D), jnp.float32),
    )(pm, cnt, W1, b1.reshape(1, D), W2, b2.reshape(1, D))


def kernel(x, W1, b1, W2, b2):
    entropy, cs = _entropy_cs(x)
    pm, cnt = _walk_patches(entropy, cs)
    blt = _features(pm, cnt, W1, b1, W2, b2)
    return (blt, entropy)


# speedup vs baseline: 1.6386x; 1.1202x over previous
import functools

import jax
import jax.numpy as jnp
from jax.experimental import pallas as pl
from jax.experimental.pallas import tpu as pltpu
from jax.experimental.pallas import tpu_sc as plsc

B = 8
L = 2048
D = 128
WINDOW = 9
K_SYM = 5
PATCH_HIGH = 3
PATCH_LOW = 12
ENT_THR_ROBUST = 1.51
NCAND = (L + PATCH_HIGH - 1) // PATCH_HIGH
KP = 704
CSLEN = 2080


def _ent_body(x_ref, ent_ref, cs_ref):
    x = x_ref[...]
    z4 = jnp.zeros((B, WINDOW // 2), jnp.float32)
    counts = []
    for s in range(K_SYM):
        ind = jnp.concatenate(
            [z4, (x == s).astype(jnp.float32), z4], axis=1)
        c = ind[:, 4:4 + L]
        for w in range(WINDOW):
            if w != 4:
                c = c + ind[:, w:w + L]
        counts.append(c)
    total = counts[0] + counts[1] + counts[2] + counts[3] + counts[4]
    total = jnp.maximum(total, 1e-12)
    ent = jnp.zeros((B, L), jnp.float32)
    for s in range(K_SYM):
        p = counts[s] / total
        ent = ent - p * jnp.log2(p + 1e-12)
    ent_ref[...] = ent

    xf = x.astype(jnp.float32)
    inc = xf
    sh = 1
    while sh < L:
        z = jnp.zeros((B, sh), jnp.float32)
        inc = inc + jnp.concatenate([z, inc[:, :L - sh]], axis=1)
        sh *= 2
    zcol = jnp.zeros((B, 1), jnp.float32)
    ztail = jnp.zeros((B, CSLEN - L - 1), jnp.float32)
    cs_ref[...] = jnp.concatenate([zcol, inc, ztail], axis=1)


def _entropy_cs(x):
    return pl.pallas_call(
        _ent_body,
        out_shape=[
            jax.ShapeDtypeStruct((B, L), jnp.float32),
            jax.ShapeDtypeStruct((B, CSLEN), jnp.float32),
        ],
    )(x)


def _walk_patches(entropy, cs):
    mesh = plsc.VectorSubcoreMesh(core_axis_name="c", subcore_axis_name="s",
                                  num_cores=1)
    nworkers = 16

    @functools.partial(
        pl.kernel,
        out_type=[
            jax.ShapeDtypeStruct((B, KP), jnp.float32),
            jax.ShapeDtypeStruct((B, 16), jnp.float32),
        ],
        mesh=mesh,
        scratch_types=[
            pltpu.VMEM((L + 64,), jnp.float32),
            pltpu.VMEM((CSLEN,), jnp.float32),
            pltpu.VMEM((KP,), jnp.float32),
            pltpu.VMEM((16,), jnp.float32),
        ],
    )
    def walk(ent_hbm, cs_hbm, pm_hbm, cnt_hbm, ent_v, cs_v, pm_v, cnt_v):
        wid = jax.lax.axis_index("s") + jax.lax.axis_index("c")
        row = jax.lax.rem(wid, B)
        pltpu.sync_copy(ent_hbm.at[row], ent_v.at[pl.ds(0, L)])
        pltpu.sync_copy(cs_hbm.at[row], cs_v)
        zero16 = jnp.zeros((16,), jnp.float32)
        ent_v[pl.ds(L, 16)] = zero16
        ent_v[pl.ds(L + 16, 16)] = zero16
        ent_v[pl.ds(L + 32, 16)] = zero16
        ent_v[pl.ds(L + 48, 16)] = zero16
        lane_i = jax.lax.iota(jnp.int32, 16)
        lane0_f = (1 - jnp.minimum(lane_i, 1)).astype(jnp.float32)
        d1 = lane_i - 1
        lane1_f = (1 - jnp.minimum(d1 * d1, 1)).astype(jnp.float32)

        def body(_, carry):
            i, t, last, e_cur, cs_cur = carry
            act_a = i < L
            act_af = jnp.where(act_a, 1.0, 0.0)
            hi_a = e_cur > ENT_THR_ROBUST
            ja3 = jnp.minimum(i + 3, L)
            ja12 = jnp.minimum(i + 12, L)
            j6 = jnp.minimum(i + 6, L)
            j15 = jnp.minimum(i + 15, L)
            j24 = jnp.minimum(i + 24, L)
            eA3 = ent_v[pl.ds(i + 3, 16)][0]
            eA12 = ent_v[pl.ds(i + 12, 16)][0]
            cA3 = cs_v[pl.ds(ja3, 16)][0]
            cA12 = cs_v[pl.ds(ja12, 16)][0]
            eB6 = ent_v[pl.ds(i + 6, 16)][0]
            eB15 = ent_v[pl.ds(i + 15, 16)][0]
            eB24 = ent_v[pl.ds(i + 24, 16)][0]
            cB6 = cs_v[pl.ds(j6, 16)][0]
            cB15 = cs_v[pl.ds(j15, 16)][0]
            cB24 = cs_v[pl.ds(j24, 16)][0]
            jA = jnp.where(hi_a, ja3, ja12)
            csA = jnp.where(hi_a, cA3, cA12)
            rdenA = jnp.where(hi_a, 1.0 / PATCH_HIGH, 1.0 / PATCH_LOW)
            pmA = (csA - cs_cur) * rdenA
            iA = jnp.where(act_a, jA, i)
            lastA = jnp.where(act_a, i, last)
            eA = jnp.where(act_a, jnp.where(hi_a, eA3, eA12), e_cur)
            csA2 = jnp.where(act_a, csA, cs_cur)
            tA = t + jnp.where(act_a, 1, 0)
            act_b = iA < L
            hi_b = eA > ENT_THR_ROBUST
            jB = jnp.where(hi_b, jnp.where(hi_a, j6, j15),
                           jnp.where(hi_a, j15, j24))
            csB = jnp.where(hi_b, jnp.where(hi_a, cB6, cB15),
                            jnp.where(hi_a, cB15, cB24))
            eBv = jnp.where(hi_b, jnp.where(hi_a, eB6, eB15),
                            jnp.where(hi_a, eB15, eB24))
            rdenB = jnp.where(hi_b, 1.0 / PATCH_HIGH, 1.0 / PATCH_LOW)
            pmB = (csB - csA2) * rdenB
            iB = jnp.where(act_b, jB, iA)
            lastB = jnp.where(act_b, iA, lastA)
            eF = jnp.where(act_b, eBv, eA)
            csF = jnp.where(act_b, csB, csA2)
            tB = tA + jnp.where(act_b, 1, 0)
            laneB_f = lane1_f * act_af + lane0_f * (1.0 - act_af)
            pm_v[pl.ds(t, 16)] = lane0_f * pmA + laneB_f * pmB
            return (iB, tB, lastB, eF, csF)

        e0 = ent_v[pl.ds(0, 16)][0]
        init = (jnp.int32(0), jnp.int32(0), jnp.int32(0), e0,
                jnp.float32(0.0))
        final = jax.lax.fori_loop(0, NCAND // 2 + 1, body, init, unroll=4)
        count = final[1]
        i_last = final[2]
        cs_total = cs_v[pl.ds(L, 16)][0]
        cs_il = cs_v[pl.ds(i_last, 16)][0]
        def lane(k):
            dk = lane_i - k
            return (1 - jnp.minimum(dk * dk, 1)).astype(jnp.float32)
        cnt_v[...] = (lane(0) * count.astype(jnp.float32)
                      + lane(1) * i_last.astype(jnp.float32)
                      + lane(2) * cs_total + lane(3) * cs_il)

        @pl.when(wid < B)
        def _():
            pltpu.sync_copy(pm_v, pm_hbm.at[wid])
            pltpu.sync_copy(cnt_v, cnt_hbm.at[wid])

    return walk(entropy, cs)


def _feat_body(pm_ref, cnt_ref, w1_ref, b1_ref, w2_ref, b2_ref, out_ref):
    pm = pm_ref[...]
    count = cnt_ref[:, 0:1]
    i_last = cnt_ref[:, 1:2]
    cs_total = cnt_ref[:, 2:3]
    cs_il = cnt_ref[:, 3:4]
    den_last = jnp.maximum(float(L) - i_last, 1.0)
    pm_last = (cs_total - cs_il) / den_last
    tt = jax.lax.broadcasted_iota(jnp.int32, (B, KP), 1).astype(jnp.float32)
    pm = jnp.where(tt == count - 1.0, pm_last, pm)
    msk = (tt < count).astype(jnp.float32)
    w1 = w1_ref[...]
    b1 = b1_ref[...]
    h = jnp.maximum(pm[:, :, None] * w1 + b1, 0.0) * msk[:, :, None]
    s_h = jnp.sum(h, axis=1)
    out = jax.lax.dot_general(
        s_h, w2_ref[...], (((1,), (0,)), ((), ())),
        preferred_element_type=jnp.float32,
    )
    out_ref[...] = out / count + b2_ref[...]


def _features(pm, cnt, W1, b1, W2, b2):
    return pl.pallas_call(
        _feat_body,
        out_shape=jax.ShapeDtypeStruct((B, D), jnp.float32),
    )(pm, cnt, W1, b1.reshape(1, D), W2, b2.reshape(1, D))


def kernel(x, W1, b1, W2, b2):
    entropy, cs = _entropy_cs(x)
    pm, cnt = _walk_patches(entropy, cs)
    blt = _features(pm, cnt, W1, b1, W2, b2)
    return (blt, entropy)
